# baseline (speedup 1.0000x reference)
import functools

import jax
import jax.numpy as jnp
from jax import lax
from jax.experimental import pallas as pl
from jax.experimental.pallas import tpu as pltpu
from jax.experimental.pallas import tpu_sc as plsc

_N = 10000
_D = 128
_E = 320000
_NC = 2
_NS = 16
_HW = _D // 2
_PW = _HW // 2
_EPT = _E // _NS
_CH = 128
_NCHUNK = 160
_PAD = _NCHUNK * _CH - _EPT
_SB = 40
_NSB = _NCHUNK // _SB
_NP = 10240
_RPT = _NP // _NS

_mesh = plsc.VectorSubcoreMesh(core_axis_name="c", subcore_axis_name="s")

_scratch = [
    pltpu.VMEM((_SB, _CH), jnp.int32),
    pltpu.VMEM((_SB, _CH), jnp.int32),
    pltpu.VMEM((_CH, _PW), jnp.int32),
    pltpu.VMEM((_CH, _PW), jnp.int32),
    pltpu.VMEM((_CH, _HW), jnp.float32),
    pltpu.VMEM((_CH, _HW), jnp.float32),
    pltpu.VMEM_SHARED((_NP, _HW), jnp.float32),
    pltpu.SemaphoreType.DMA,
    pltpu.SemaphoreType.DMA,
    pltpu.SemaphoreType.DMA,
    pltpu.SemaphoreType.DMA,
]

_HIMASK = jnp.int32(-65536)


def _unpack_chunk(pk, fr):
    def row(r, carry):
        for g in range(_PW // 16):
            v = pk[r, pl.ds(16 * g, 16)]
            fr[r, pl.ds(16 * g, 16)] = plsc.bitcast(v << 16, jnp.float32)
            fr[r, pl.ds(_PW + 16 * g, 16)] = plsc.bitcast(v & _HIMASK,
                                                          jnp.float32)
        return carry

    lax.fori_loop(0, _CH, row, 0)


def _zero_acc(zero_hbm, fr1, aggsh, r0, gsem):
    pltpu.sync_copy(zero_hbm, fr1)
    for m in range(_RPT // _CH):
        pltpu.async_copy(fr1, aggsh.at[pl.ds(r0 + m * _CH, _CH)], gsem)
    for m in range(_RPT // _CH):
        pltpu.make_async_copy(fr1, aggsh.at[pl.ds(r0, _CH)], gsem).wait()


def _read_acc(aggsh, fr1, out_hbm, c, r0):
    for m in range(_RPT // _CH):
        pltpu.sync_copy(aggsh.at[pl.ds(r0 + m * _CH, _CH)], fr1)
        pltpu.sync_copy(fr1, out_hbm.at[c].at[pl.ds(r0 + m * _CH, _CH)])


def _edge_pass(h2_hbm, src_hbm, dst_hbm, c, s, srcb, dstb, pk, fr, gsems,
               ssems, aggsh):
    for sb in range(_NSB):
        pltpu.sync_copy(src_hbm.at[c, s, pl.ds(sb * _SB, _SB)], srcb)
        pltpu.sync_copy(dst_hbm.at[s, pl.ds(sb * _SB, _SB)], dstb)
        pltpu.async_copy(h2_hbm.at[srcb.at[0]], pk[0], gsems[0])

        def step(g, carry):
            for b in range(2):
                j = 2 * g + b
                pltpu.make_async_copy(h2_hbm.at[srcb.at[0]], pk[b],
                                      gsems[b]).wait()

                @pl.when(j + 1 < _SB)
                def _():
                    pltpu.async_copy(h2_hbm.at[srcb.at[j + 1]], pk[1 - b],
                                     gsems[1 - b])

                @pl.when(j >= 2)
                def _():
                    pltpu.make_async_copy(fr[b], aggsh.at[dstb.at[0]],
                                          ssems[b]).wait()

                _unpack_chunk(pk[b], fr[b])
                pltpu.async_copy(fr[b], aggsh.at[dstb.at[j]], ssems[b],
                                 add=True)
            return carry

        lax.fori_loop(0, _SB // 2, step, 0)
        pltpu.make_async_copy(fr[0], aggsh.at[dstb.at[0]], ssems[0]).wait()
        pltpu.make_async_copy(fr[1], aggsh.at[dstb.at[0]], ssems[1]).wait()


@functools.partial(
    pl.kernel,
    out_type=jax.ShapeDtypeStruct((_NC, _NP, _HW), jnp.float32),
    mesh=_mesh,
    scratch_types=_scratch,
    compiler_params=pltpu.CompilerParams(needs_layout_passes=False,
                                         use_tc_tiling_on_sc=False),
)
def _sc_agg(h2_hbm, src_hbm, dst_hbm, zero_hbm, out_hbm,
            srcb, dstb, pk0, pk1, fr0, fr1, aggsh, gsem0, gsem1, ssem0, ssem1):
    c = lax.axis_index("c")
    s = lax.axis_index("s")
    r0 = s * _RPT
    _zero_acc(zero_hbm, fr1, aggsh, r0, gsem0)
    plsc.subcore_barrier()
    _edge_pass(h2_hbm, src_hbm, dst_hbm, c, s, srcb, dstb, (pk0, pk1),
               (fr0, fr1), (gsem0, gsem1), (ssem0, ssem1), aggsh)
    plsc.subcore_barrier()
    _read_acc(aggsh, fr1, out_hbm, c, r0)


@functools.partial(
    pl.kernel,
    out_type=(jax.ShapeDtypeStruct((_NC, _NP, _HW), jnp.float32),
              jax.ShapeDtypeStruct((_NC, _NP, _HW), jnp.float32)),
    mesh=_mesh,
    scratch_types=_scratch,
    compiler_params=pltpu.CompilerParams(needs_layout_passes=False,
                                         use_tc_tiling_on_sc=False),
)
def _sc_agg0(h2_hbm, src_hbm, dst_hbm, zero_hbm, ones_hbm, out_hbm, deg_hbm,
             srcb, dstb, pk0, pk1, fr0, fr1, aggsh, gsem0, gsem1, ssem0,
             ssem1):
    c = lax.axis_index("c")
    s = lax.axis_index("s")
    r0 = s * _RPT
    _zero_acc(zero_hbm, fr1, aggsh, r0, gsem0)
    plsc.subcore_barrier()
    _edge_pass(h2_hbm, src_hbm, dst_hbm, c, s, srcb, dstb, (pk0, pk1),
               (fr0, fr1), (gsem0, gsem1), (ssem0, ssem1), aggsh)
    plsc.subcore_barrier()
    _read_acc(aggsh, fr1, out_hbm, c, r0)
    _zero_acc(zero_hbm, fr1, aggsh, r0, gsem0)
    pltpu.sync_copy(ones_hbm, fr0)
    plsc.subcore_barrier()
    for k in range(_NSB // _NC):
        sbx = c * (_NSB // _NC) + k
        pltpu.sync_copy(dst_hbm.at[s, pl.ds(sbx * _SB, _SB)], dstb)
        for j0 in range(4):
            pltpu.async_copy(fr0, aggsh.at[dstb.at[j0]], ssem0, add=True)

        def dstep(g, carry):
            pltpu.make_async_copy(fr0, aggsh.at[dstb.at[0]], ssem0).wait()
            pltpu.async_copy(fr0, aggsh.at[dstb.at[g + 4]], ssem0, add=True)
            return carry

        lax.fori_loop(0, _SB - 4, dstep, 0)
        for _ in range(4):
            pltpu.make_async_copy(fr0, aggsh.at[dstb.at[0]], ssem0).wait()
    plsc.subcore_barrier()
    _read_acc(aggsh, fr1, deg_hbm, c, r0)


_BLK = 2000


def _pack_rows(acc):
    def rne16(z):
        y = lax.bitcast_convert_type(z, jnp.int32)
        return (y + 0x7FFF + ((y >> 16) & 1)) >> 16

    slabs = []
    for cc in range(_NC):
        lo = rne16(acc[:, cc * _HW:cc * _HW + _PW])
        hi = rne16(acc[:, cc * _HW + _PW:(cc + 1) * _HW])
        slabs.append((lo & 0xFFFF) | (hi << 16))
    return slabs


def _tc0_body(h_ref, p_ref, d_ref, ws_ref, wn_ref, b_ref,
              o_ref, o2_ref, iv_ref):
    deg = d_ref[0, :, 0:1] + d_ref[1, :, 0:1]
    inv = 1.0 / jnp.maximum(deg, 1.0)
    iv_ref[...] = jnp.broadcast_to(inv, (_BLK, 8))
    mean = jnp.concatenate([p_ref[0], p_ref[1]], axis=1) * inv
    acc = jnp.dot(h_ref[...], ws_ref[...], preferred_element_type=jnp.float32)
    acc = acc + jnp.dot(mean, wn_ref[...], preferred_element_type=jnp.float32)
    acc = jnp.maximum(acc + b_ref[...], 0.0)
    o_ref[...] = acc
    lo, hi = _pack_rows(acc)
    o2_ref[0] = lo
    o2_ref[1] = hi


def _tc_body(h_ref, p_ref, iv_ref, ws_ref, wn_ref, b_ref, o_ref, o2_ref):
    mean = jnp.concatenate([p_ref[0], p_ref[1]], axis=1) * iv_ref[:, 0:1]
    acc = jnp.dot(h_ref[...], ws_ref[...], preferred_element_type=jnp.float32)
    acc = acc + jnp.dot(mean, wn_ref[...], preferred_element_type=jnp.float32)
    acc = jnp.maximum(acc + b_ref[...], 0.0)
    o_ref[...] = acc
    lo, hi = _pack_rows(acc)
    o2_ref[0] = lo
    o2_ref[1] = hi


def _tc_last_body(h_ref, p_ref, iv_ref, ws_ref, wn_ref, b_ref, o_ref):
    mean = jnp.concatenate([p_ref[0], p_ref[1]], axis=1) * iv_ref[:, 0:1]
    acc = jnp.dot(h_ref[...], ws_ref[...], preferred_element_type=jnp.float32)
    acc = acc + jnp.dot(mean, wn_ref[...], preferred_element_type=jnp.float32)
    o_ref[...] = jnp.maximum(acc + b_ref[...], 0.0)


_h_spec = pl.BlockSpec((_BLK, _D), lambda i: (i, 0))
_p_spec = pl.BlockSpec((_NC, _BLK, _HW), lambda i: (0, i, 0))
_iv_spec = pl.BlockSpec((_BLK, 8), lambda i: (i, 0))
_w_spec = pl.BlockSpec((_D, _D), lambda i: (0, 0))
_b_spec = pl.BlockSpec((1, _D), lambda i: (0, 0))
_o2_spec = pl.BlockSpec((_NC, _BLK, _PW), lambda i: (0, i, 0))

_tc_layer0 = pl.pallas_call(
    _tc0_body,
    grid=(_N // _BLK,),
    in_specs=[_h_spec, _p_spec, _p_spec, _w_spec, _w_spec, _b_spec],
    out_specs=[_h_spec, _o2_spec, _iv_spec],
    out_shape=[
        jax.ShapeDtypeStruct((_N, _D), jnp.float32),
        jax.ShapeDtypeStruct((_NC, _N, _PW), jnp.int32),
        jax.ShapeDtypeStruct((_N, 8), jnp.float32),
    ],
)

_tc_layer1 = pl.pallas_call(
    _tc_body,
    grid=(_N // _BLK,),
    in_specs=[_h_spec, _p_spec, _iv_spec, _w_spec, _w_spec, _b_spec],
    out_specs=[_h_spec, _o2_spec],
    out_shape=[
        jax.ShapeDtypeStruct((_N, _D), jnp.float32),
        jax.ShapeDtypeStruct((_NC, _N, _PW), jnp.int32),
    ],
)

_tc_layer2 = pl.pallas_call(
    _tc_last_body,
    grid=(_N // _BLK,),
    in_specs=[_h_spec, _p_spec, _iv_spec, _w_spec, _w_spec, _b_spec],
    out_specs=_h_spec,
    out_shape=jax.ShapeDtypeStruct((_N, _D), jnp.float32),
)


def kernel(x, edge_index, W_self_0, W_neigh_0, b_0, W_self_1, W_neigh_1, b_1,
           W_self_2, W_neigh_2, b_2):
    pad_src = (jnp.arange(_PAD, dtype=jnp.int32) * 41) % _N
    pad_dst = _N + jnp.arange(_PAD, dtype=jnp.int32) % (_NP - _N)
    srct = jnp.concatenate(
        [edge_index[0].reshape(_NS, _EPT),
         jnp.broadcast_to(pad_src, (_NS, _PAD))], axis=1)
    srcx = jnp.stack([srct, srct + _N]).reshape(_NC, _NS, _NCHUNK, _CH)
    dstx = jnp.concatenate(
        [edge_index[1].reshape(_NS, _EPT),
         jnp.broadcast_to(pad_dst, (_NS, _PAD))], axis=1
    ).reshape(_NS, _NCHUNK, _CH)
    zeros = jnp.zeros((_CH, _HW), jnp.float32)
    ones = jnp.ones((_CH, _HW), jnp.float32)

    x16 = x.astype(jnp.bfloat16)
    x2 = jnp.stack([
        lax.bitcast_convert_type(
            jnp.stack([x16[:, cc * _HW:cc * _HW + _PW],
                       x16[:, cc * _HW + _PW:(cc + 1) * _HW]], axis=-1),
            jnp.int32)
        for cc in range(_NC)
    ]).reshape(_NC * _N, _PW)

    parts, degp = _sc_agg0(x2, srcx, dstx, zeros, ones)
    h, h2, invd = _tc_layer0(x, parts, degp, W_self_0, W_neigh_0,
                             b_0.reshape(1, _D))
    parts = _sc_agg(h2.reshape(_NC * _N, _PW), srcx, dstx, zeros)
    h, h2 = _tc_layer1(h, parts, invd, W_self_1, W_neigh_1, b_1.reshape(1, _D))
    parts = _sc_agg(h2.reshape(_NC * _N, _PW), srcx, dstx, zeros)
    h = _tc_layer2(h, parts, invd, W_self_2, W_neigh_2, b_2.reshape(1, _D))
    return h.reshape(1, _N, _D)


# speedup vs baseline: 1.5198x; 1.5198x over previous
import functools

import jax
import jax.numpy as jnp
from jax import lax
from jax.experimental import pallas as pl
from jax.experimental.pallas import tpu as pltpu
from jax.experimental.pallas import tpu_sc as plsc

_N = 10000
_D = 128
_E = 320000
_NC = 2
_NS = 16
_NW = _NC * _NS
_EPW = _E // _NW
_CH = 128
_NCHUNK = 80
_PAD = _NCHUNK * _CH - _EPW
_SB = 40
_NSB = _NCHUNK // _SB
_NP = 10240
_RPT = _NP // _NS

_mesh = plsc.VectorSubcoreMesh(core_axis_name="c", subcore_axis_name="s")

_scratch = [
    pltpu.VMEM((_SB, _CH), jnp.int32),
    pltpu.VMEM((_SB, _CH), jnp.int32),
    pltpu.VMEM((_CH, _D), jnp.float32),
    pltpu.VMEM((_CH, _D), jnp.float32),
    pltpu.VMEM_SHARED((_NP, _D), jnp.float32),
    pltpu.SemaphoreType.DMA,
    pltpu.SemaphoreType.DMA,
    pltpu.SemaphoreType.DMA,
    pltpu.SemaphoreType.DMA,
]


def _edge_pass(h_hbm, src_hbm, dst_hbm, wid, srcb, dstb, rows, gsems, ssems,
               aggsh):
    for sb in range(_NSB):
        pltpu.sync_copy(src_hbm.at[wid, pl.ds(sb * _SB, _SB)], srcb)
        pltpu.sync_copy(dst_hbm.at[wid, pl.ds(sb * _SB, _SB)], dstb)
        pltpu.async_copy(h_hbm.at[srcb.at[0]], rows[0], gsems[0])

        def step(g, carry):
            for b in range(2):
                j = 2 * g + b
                pltpu.make_async_copy(h_hbm.at[srcb.at[0]], rows[b],
                                      gsems[b]).wait()
                pltpu.async_copy(rows[b], aggsh.at[dstb.at[j]], ssems[b],
                                 add=True)

                @pl.when(j >= 1)
                def _():
                    pltpu.make_async_copy(rows[1 - b], aggsh.at[dstb.at[0]],
                                          ssems[1 - b]).wait()

                @pl.when(j + 1 < _SB)
                def _():
                    pltpu.async_copy(h_hbm.at[srcb.at[j + 1]], rows[1 - b],
                                     gsems[1 - b])

            return carry

        lax.fori_loop(0, _SB // 2, step, 0)
        pltpu.make_async_copy(rows[(_SB - 1) % 2], aggsh.at[dstb.at[0]],
                              ssems[(_SB - 1) % 2]).wait()


@functools.partial(
    pl.kernel,
    out_type=jax.ShapeDtypeStruct((_NC, _NP, _D), jnp.float32),
    mesh=_mesh,
    scratch_types=_scratch,
)
def _sc_agg(h_hbm, src_hbm, dst_hbm, zero_hbm, out_hbm,
            srcb, dstb, rows0, rows1, aggsh, gsem0, gsem1, ssem0, ssem1):
    c = lax.axis_index("c")
    s = lax.axis_index("s")
    wid = c * _NS + s
    r0 = s * _RPT
    pltpu.sync_copy(zero_hbm.at[pl.ds(r0, _RPT)], aggsh.at[pl.ds(r0, _RPT)])
    plsc.subcore_barrier()
    _edge_pass(h_hbm, src_hbm, dst_hbm, wid, srcb, dstb, (rows0, rows1),
               (gsem0, gsem1), (ssem0, ssem1), aggsh)
    plsc.subcore_barrier()
    pltpu.sync_copy(aggsh.at[pl.ds(r0, _RPT)], out_hbm.at[c].at[pl.ds(r0, _RPT)])


@functools.partial(
    pl.kernel,
    out_type=(jax.ShapeDtypeStruct((_NC, _NP, _D), jnp.float32),
              jax.ShapeDtypeStruct((_NC, _NP, _D), jnp.float32)),
    mesh=_mesh,
    scratch_types=_scratch,
)
def _sc_agg0(h_hbm, src_hbm, dst_hbm, zero_hbm, ones_hbm, out_hbm, deg_hbm,
             srcb, dstb, rows0, rows1, aggsh, gsem0, gsem1, ssem0, ssem1):
    c = lax.axis_index("c")
    s = lax.axis_index("s")
    wid = c * _NS + s
    r0 = s * _RPT
    pltpu.sync_copy(zero_hbm.at[pl.ds(r0, _RPT)], aggsh.at[pl.ds(r0, _RPT)])
    plsc.subcore_barrier()
    _edge_pass(h_hbm, src_hbm, dst_hbm, wid, srcb, dstb, (rows0, rows1),
               (gsem0, gsem1), (ssem0, ssem1), aggsh)
    plsc.subcore_barrier()
    pltpu.sync_copy(aggsh.at[pl.ds(r0, _RPT)], out_hbm.at[c].at[pl.ds(r0, _RPT)])
    pltpu.sync_copy(zero_hbm.at[pl.ds(r0, _RPT)], aggsh.at[pl.ds(r0, _RPT)])
    pltpu.sync_copy(ones_hbm, rows0)
    plsc.subcore_barrier()
    for sb in range(_NSB - 1, -1, -1):
        if sb != _NSB - 1:
            pltpu.sync_copy(dst_hbm.at[wid, pl.ds(sb * _SB, _SB)], dstb)
        for j0 in range(4):
            pltpu.async_copy(rows0, aggsh.at[dstb.at[j0]], ssem0, add=True)

        def dstep(g, carry):
            pltpu.make_async_copy(rows0, aggsh.at[dstb.at[0]], ssem0).wait()
            pltpu.async_copy(rows0, aggsh.at[dstb.at[g + 4]], ssem0, add=True)
            return carry

        lax.fori_loop(0, _SB - 4, dstep, 0)
        for _ in range(4):
            pltpu.make_async_copy(rows0, aggsh.at[dstb.at[0]], ssem0).wait()
    plsc.subcore_barrier()
    pltpu.sync_copy(aggsh.at[pl.ds(r0, _RPT)], deg_hbm.at[c].at[pl.ds(r0, _RPT)])


_BLK = 2000


def _tc0_body(h_ref, p_ref, d_ref, ws_ref, wn_ref, b_ref, o_ref, iv_ref):
    deg = d_ref[0, :, 0:1] + d_ref[1, :, 0:1]
    inv = 1.0 / jnp.maximum(deg, 1.0)
    iv_ref[...] = jnp.broadcast_to(inv, (_BLK, 8))
    mean = (p_ref[0] + p_ref[1]) * inv
    acc = jnp.dot(h_ref[...], ws_ref[...], preferred_element_type=jnp.float32)
    acc = acc + jnp.dot(mean, wn_ref[...], preferred_element_type=jnp.float32)
    o_ref[...] = jnp.maximum(acc + b_ref[...], 0.0)


_tc_layer0 = pl.pallas_call(
    _tc0_body,
    grid=(_N // _BLK,),
    in_specs=[
        pl.BlockSpec((_BLK, _D), lambda i: (i, 0)),
        pl.BlockSpec((_NC, _BLK, _D), lambda i: (0, i, 0)),
        pl.BlockSpec((_NC, _BLK, _D), lambda i: (0, i, 0)),
        pl.BlockSpec((_D, _D), lambda i: (0, 0)),
        pl.BlockSpec((_D, _D), lambda i: (0, 0)),
        pl.BlockSpec((1, _D), lambda i: (0, 0)),
    ],
    out_specs=[
        pl.BlockSpec((_BLK, _D), lambda i: (i, 0)),
        pl.BlockSpec((_BLK, 8), lambda i: (i, 0)),
    ],
    out_shape=[
        jax.ShapeDtypeStruct((_N, _D), jnp.float32),
        jax.ShapeDtypeStruct((_N, 8), jnp.float32),
    ],
)


def _tc_body(h_ref, p_ref, iv_ref, ws_ref, wn_ref, b_ref, o_ref):
    mean = (p_ref[0] + p_ref[1]) * iv_ref[:, 0:1]
    acc = jnp.dot(h_ref[...], ws_ref[...], preferred_element_type=jnp.float32)
    acc = acc + jnp.dot(mean, wn_ref[...], preferred_element_type=jnp.float32)
    o_ref[...] = jnp.maximum(acc + b_ref[...], 0.0)


_tc_layer = pl.pallas_call(
    _tc_body,
    grid=(_N // _BLK,),
    in_specs=[
        pl.BlockSpec((_BLK, _D), lambda i: (i, 0)),
        pl.BlockSpec((_NC, _BLK, _D), lambda i: (0, i, 0)),
        pl.BlockSpec((_BLK, 8), lambda i: (i, 0)),
        pl.BlockSpec((_D, _D), lambda i: (0, 0)),
        pl.BlockSpec((_D, _D), lambda i: (0, 0)),
        pl.BlockSpec((1, _D), lambda i: (0, 0)),
    ],
    out_specs=pl.BlockSpec((_BLK, _D), lambda i: (i, 0)),
    out_shape=jax.ShapeDtypeStruct((_N, _D), jnp.float32),
)


def kernel(x, edge_index, W_self_0, W_neigh_0, b_0, W_self_1, W_neigh_1, b_1,
           W_self_2, W_neigh_2, b_2):
    pad_src = (jnp.arange(_PAD, dtype=jnp.int32) * 41) % _N
    pad_dst = _N + jnp.arange(_PAD, dtype=jnp.int32)
    src = jnp.concatenate(
        [edge_index[0].reshape(_NW, _EPW),
         jnp.broadcast_to(pad_src, (_NW, _PAD))], axis=1
    ).reshape(_NW, _NCHUNK, _CH)
    dst = jnp.concatenate(
        [edge_index[1].reshape(_NW, _EPW),
         jnp.broadcast_to(pad_dst, (_NW, _PAD))], axis=1
    ).reshape(_NW, _NCHUNK, _CH)
    zeros = jnp.zeros((_NP, _D), jnp.float32)
    ones = jnp.ones((_CH, _D), jnp.float32)

    parts, degp = _sc_agg0(x, src, dst, zeros, ones)
    params = [(W_self_1, W_neigh_1, b_1), (W_self_2, W_neigh_2, b_2)]
    h, invd = _tc_layer0(x, parts, degp, W_self_0, W_neigh_0,
                         b_0.reshape(1, _D))
    for Ws, Wn, b in params:
        parts = _sc_agg(h, src, dst, zeros)
        h = _tc_layer(h, parts, invd, Ws, Wn, b.reshape(1, _D))
    return h.reshape(1, _N, _D)
